# baseline (speedup 1.0000x reference)
import functools

import jax
import jax.numpy as jnp
from jax import lax
from jax.experimental import pallas as pl
from jax.experimental.pallas import tpu as pltpu
from jax.experimental.pallas import tpu_sc as plsc

B, S, D = 1, 2048, 1024
E, K, H = 8, 2, 1024
N = B * S
TT = 256
NPAD = N * K + E * TT
NT2 = NPAD // TT
NEG = -1e30

NW = 32
GCH = 64
CCH = 32


def _router_body(wg_ref, x_ref, logits_ref, sel_ref, rw_ref):
    xt = x_ref[...]
    lt = lax.dot_general(
        wg_ref[...], xt, (((1,), (1,)), ((), ())),
        preferred_element_type=jnp.float32)
    idx = lax.broadcasted_iota(jnp.int32, (E, TT), 0)
    m1 = jnp.max(lt, axis=0, keepdims=True)
    a1 = jnp.min(jnp.where(lt == m1, idx, E), axis=0, keepdims=True)
    lt2 = jnp.where(idx == a1, NEG, lt)
    m2 = jnp.max(lt2, axis=0, keepdims=True)
    a2 = jnp.min(jnp.where(lt2 == m2, idx, E), axis=0, keepdims=True)
    e2 = jnp.exp(m2 - m1)
    denom = 1.0 + e2
    logits_ref[...] = lt
    sel_ref[...] = jnp.concatenate([a1, a2], axis=0)
    rw_ref[...] = jnp.concatenate([1.0 / denom, e2 / denom], axis=0)


def _router(xf, Wg):
    nt = N // TT
    return pl.pallas_call(
        _router_body,
        grid=(nt,),
        in_specs=[
            pl.BlockSpec((E, D), lambda t: (0, 0)),
            pl.BlockSpec((TT, D), lambda t: (t, 0)),
        ],
        out_specs=[
            pl.BlockSpec((E, TT), lambda t: (0, t)),
            pl.BlockSpec((K, TT), lambda t: (0, t)),
            pl.BlockSpec((K, TT), lambda t: (0, t)),
        ],
        out_shape=[
            jax.ShapeDtypeStruct((E, N), jnp.float32),
            jax.ShapeDtypeStruct((K, N), jnp.int32),
            jax.ShapeDtypeStruct((K, N), jnp.float32),
        ],
    )(Wg, xf)


def _dispatch_meta(selT, rwT):
    sel_flat = selT.reshape(-1)
    w_flat = rwT.reshape(-1)
    tok_flat = jnp.tile(jnp.arange(N, dtype=jnp.int32), K)
    onehot = (sel_flat[:, None] == jnp.arange(E, dtype=jnp.int32)[None, :])
    oh = onehot.astype(jnp.int32)
    counts = jnp.sum(oh, axis=0)
    rank = jnp.sum((jnp.cumsum(oh, axis=0) - 1) * oh, axis=1)
    pcounts = ((counts + TT - 1) // TT) * TT
    pcum = jnp.cumsum(pcounts)
    pstart = pcum - pcounts
    dest = pstart[sel_flat] + rank
    gidx = jnp.zeros((NPAD,), jnp.int32).at[dest].set(tok_flat)
    wslot = jnp.zeros((NPAD,), jnp.float32).at[dest].set(w_flat)
    invr = dest.astype(jnp.int32).reshape(K, N)
    tile_starts = jnp.arange(NT2, dtype=jnp.int32) * TT
    te = jnp.searchsorted(pcum, tile_starts, side='right')
    active = (tile_starts < pcum[-1]).astype(jnp.int32)
    te = jnp.minimum(te, E - 1).astype(jnp.int32)
    return gidx, wslot.reshape(NPAD, 1), invr, te, active


def _sc_gather(xf, gidx):
    rpw = NPAD // NW
    mesh = plsc.VectorSubcoreMesh(core_axis_name="c", subcore_axis_name="s")

    @functools.partial(
        pl.kernel,
        out_type=jax.ShapeDtypeStruct((NPAD, D), jnp.float32),
        mesh=mesh,
        scratch_types=[
            pltpu.VMEM((GCH,), jnp.int32),
            pltpu.VMEM((GCH, D), jnp.float32),
            pltpu.SemaphoreType.DMA,
        ],
    )
    def k(x_hbm, idx_hbm, xs_hbm, idx_v, rows_v, sem):
        wid = lax.axis_index("s") * 2 + lax.axis_index("c")
        base = wid * rpw
        for c in range(rpw // GCH):
            off = base + c * GCH
            pltpu.sync_copy(idx_hbm.at[pl.ds(off, GCH)], idx_v)
            pltpu.async_copy(x_hbm.at[idx_v], rows_v, sem).wait()
            pltpu.sync_copy(rows_v, xs_hbm.at[pl.ds(off, GCH)])

    return k(xf, gidx)


def _expert_body(te_ref, act_ref, xs_ref, w1_ref, w2_ref, ws_ref, ys_ref):
    t = pl.program_id(0)

    @pl.when(act_ref[t] == 1)
    def _():
        xt = xs_ref[...]
        g = lax.dot_general(
            xt, w1_ref[0, 0], (((1,), (1,)), ((), ())),
            preferred_element_type=jnp.float32)
        l = lax.dot_general(
            xt, w1_ref[0, 1], (((1,), (1,)), ((), ())),
            preferred_element_type=jnp.float32)
        a = g * lax.logistic(g) * l
        oe = lax.dot_general(
            a, w2_ref[0], (((1,), (1,)), ((), ())),
            preferred_element_type=jnp.float32)
        ys_ref[...] = ws_ref[...] * oe

    @pl.when(act_ref[t] == 0)
    def _():
        ys_ref[...] = jnp.zeros_like(ys_ref)


def _experts(xs, W1r, W2, wslot, te, active):
    grid_spec = pltpu.PrefetchScalarGridSpec(
        num_scalar_prefetch=2,
        grid=(NT2,),
        in_specs=[
            pl.BlockSpec((TT, D), lambda t, te_r, ac_r: (t, 0)),
            pl.BlockSpec((1, 2, H, D), lambda t, te_r, ac_r: (te_r[t], 0, 0, 0)),
            pl.BlockSpec((1, D, H), lambda t, te_r, ac_r: (te_r[t], 0, 0)),
            pl.BlockSpec((TT, 1), lambda t, te_r, ac_r: (t, 0)),
        ],
        out_specs=pl.BlockSpec((TT, D), lambda t, te_r, ac_r: (t, 0)),
    )
    return pl.pallas_call(
        _expert_body,
        grid_spec=grid_spec,
        out_shape=jax.ShapeDtypeStruct((NPAD, D), jnp.float32),
    )(te, active, xs, W1r, W2, wslot)


def _sc_combine(ys, i0, i1):
    tpw = N // NW
    mesh = plsc.VectorSubcoreMesh(core_axis_name="c", subcore_axis_name="s")

    @functools.partial(
        pl.kernel,
        out_type=jax.ShapeDtypeStruct((N, D), jnp.float32),
        mesh=mesh,
        scratch_types=[
            pltpu.VMEM((CCH,), jnp.int32),
            pltpu.VMEM((CCH,), jnp.int32),
            pltpu.VMEM((CCH, D), jnp.float32),
            pltpu.VMEM((CCH, D), jnp.float32),
            pltpu.SemaphoreType.DMA,
            pltpu.SemaphoreType.DMA,
        ],
    )
    def k(ys_hbm, i0_hbm, i1_hbm, out_hbm, i0_v, i1_v, g0, g1, sem0, sem1):
        wid = lax.axis_index("s") * 2 + lax.axis_index("c")
        base = wid * tpw
        for c in range(tpw // CCH):
            off = base + c * CCH
            pltpu.sync_copy(i0_hbm.at[pl.ds(off, CCH)], i0_v)
            pltpu.sync_copy(i1_hbm.at[pl.ds(off, CCH)], i1_v)
            cp0 = pltpu.async_copy(ys_hbm.at[i0_v], g0, sem0)
            cp1 = pltpu.async_copy(ys_hbm.at[i1_v], g1, sem1)
            cp0.wait()
            cp1.wait()

            def add_row(r, carry):
                for j in range(D // 16):
                    sl = pl.ds(j * 16, 16)
                    g0[r, sl] = g0[r, sl] + g1[r, sl]
                return carry

            lax.fori_loop(0, CCH, add_row, 0)
            pltpu.sync_copy(g0, out_hbm.at[pl.ds(off, CCH)])

    return k(ys, i0, i1)


@jax.jit
def kernel(x, Wg, W1, W2):
    xf = x.reshape(N, D)
    logitsT, selT, rwT = _router(xf, Wg)
    gidx, wslot, invr, te, active = _dispatch_meta(selT, rwT)
    xs = _sc_gather(xf, gidx)
    W1r = W1.reshape(E, 2, H, D)
    ys = _experts(xs, W1r, W2, wslot, te, active)
    out = _sc_combine(ys, invr[0], invr[1])
    return out.reshape(B, S, D), logitsT.T.reshape(B, S, E)
